# baseline (speedup 1.0000x reference)
import jax
import jax.numpy as jnp
from jax.experimental import pallas as pl

_EPS = 1e-06
_H = 512
_W = 512
_N = _H * _W
_M = int(0.2 * _N)
_K = 1024


def _mask_body(member_ref, gath_ref, out_ref):
    out_ref[...] = member_ref[...] * gath_ref[...]


def kernel(cam):
    c = cam + _EPS
    s = c.sum()
    p = (c / s).reshape(_N)
    lp = jnp.log(p)

    skey = jax.random.fold_in(jax.random.key(0), 1)
    g = jax.random.gumbel(skey, (_M,), dtype=jnp.float32)

    key = jax.lax.bitcast_convert_type(p, jnp.int32)

    skeys = jnp.sort(key)
    t = skeys[_N - _M]
    n_gt = jnp.sum((key > t).astype(jnp.int32))
    need = _M - n_gt
    eq = (key == t).astype(jnp.int32)
    eqrank = jnp.cumsum(eq) - eq
    member = (key > t) | (eq.astype(bool) & (eqrank < need))

    mem_i = member.astype(jnp.int32)
    k_arr = jnp.cumsum(mem_i) - mem_i

    lv = -jnp.sort(-lp)
    score = lv[:_M] + g

    sscore = -jnp.sort(-score)
    tau = sscore[_K - 1]
    n2 = jnp.sum((score > tau).astype(jnp.int32))
    need2 = _K - n2
    eq2 = (score == tau).astype(jnp.int32)
    eqrank2 = jnp.cumsum(eq2) - eq2
    sel = ((score > tau) | (eq2.astype(bool) & (eqrank2 < need2))).astype(jnp.int32)
    sel_pad = jnp.concatenate([sel, jnp.zeros((4,), jnp.int32)])

    k_safe = jnp.where(member, k_arr, _M)
    gathered = sel_pad[k_safe]

    out = pl.pallas_call(
        _mask_body,
        out_shape=jax.ShapeDtypeStruct((_H, _W), jnp.int32),
    )(mem_i.reshape(_H, _W), gathered.reshape(_H, _W))
    return out


# speedup vs baseline: 7.4848x; 7.4848x over previous
import functools

import jax
import jax.numpy as jnp
from jax import lax
from jax.experimental import pallas as pl
from jax.experimental.pallas import tpu as pltpu
from jax.experimental.pallas import tpu_sc as plsc

_EPS = 1e-06
_H = 512
_W = 512
_N = _H * _W
_M = int(0.2 * _N)
_K = 1024
_R = 2048
_L = 128
_IMIN = -2147483648


def _partner(x, d, lane, row):
    if d < _L:
        up = jnp.roll(x, -d, axis=1)
        dn = jnp.roll(x, d, axis=1)
        bit = (lane & d) == 0
    else:
        dr = d // _L
        up = jnp.roll(x, -dr, axis=0)
        dn = jnp.roll(x, dr, axis=0)
        bit = (row & dr) == 0
    return jnp.where(bit, up, dn)


def _bitonic_desc(key, payload, lane, row):
    for s in range(1, 19):
        if s < 7:
            dirbit = (lane & (1 << s)) != 0
        elif s < 18:
            dirbit = (row & (1 << (s - 7))) != 0
        else:
            dirbit = None
        for e in range(s - 1, -1, -1):
            d = 1 << e
            pk = _partner(key, d, lane, row)
            if d < _L:
                upper = (lane & d) != 0
            else:
                upper = (row & (d // _L)) != 0
            if dirbit is None:
                keep_max = jnp.logical_not(upper)
            else:
                keep_max = upper == dirbit
            take = (keep_max & (pk > key)) | (jnp.logical_not(keep_max) & (pk < key))
            key = jnp.where(take, pk, key)
            if payload is not None:
                pp = _partner(payload, d, lane, row)
                payload = jnp.where(take, pp, payload)
    return key, payload


def _excl_cumsum(x_f32, tri_l):
    pre = jax.lax.dot(x_f32, tri_l, preferred_element_type=jnp.float32)
    rs = jnp.sum(x_f32, axis=1, keepdims=True)
    row1 = lax.broadcasted_iota(jnp.int32, (_R, 1), 0)
    c = rs
    for e in range(11):
        d = 1 << e
        c = c + jnp.where(row1 >= d, jnp.roll(c, d, axis=0), 0.0)
    return pre + (c - rs)


def _extract(x, r0, l0, lane, row):
    m = (row == r0) & (lane == l0)
    return jnp.sum(jnp.where(m, x, 0))


def _k1a_body(key_ref, lp_ref, g_ref, o_ref, ksafe_ref):
    lane = lax.broadcasted_iota(jnp.int32, (_R, _L), 1)
    row = lax.broadcasted_iota(jnp.int32, (_R, _L), 0)
    flat = row * _L + lane
    tri_l = (lax.broadcasted_iota(jnp.int32, (_L, _L), 0)
             < lax.broadcasted_iota(jnp.int32, (_L, _L), 1)).astype(jnp.float32)

    key = key_ref[...]
    lp = lp_ref[...]
    g = g_ref[...]

    ks, lps = _bitonic_desc(key, lp, lane, row)

    t = _extract(ks, (_M - 1) // _L, (_M - 1) % _L, lane, row)
    n_gt = jnp.sum(jnp.where(key > t, 1.0, 0.0))
    need = _M - n_gt.astype(jnp.int32)
    eq = key == t
    eqrank = _excl_cumsum(jnp.where(eq, 1.0, 0.0), tri_l).astype(jnp.int32)
    member = (key > t) | (eq & (eqrank < need))

    k_arr = _excl_cumsum(jnp.where(member, 1.0, 0.0), tri_l).astype(jnp.int32)

    valid = flat < _M
    score = lps + g
    o = lax.bitcast_convert_type(score, jnp.int32)
    o = o ^ (lax.shift_right_arithmetic(o, 31) & 0x7FFFFFFF)
    o_ref[...] = jnp.where(valid, o, jnp.int32(_IMIN))
    ksafe_ref[...] = jnp.where(member, k_arr, _M + (flat & 8191))


def _k1b_body(o_in_ref, sel_ref):
    lane = lax.broadcasted_iota(jnp.int32, (_R, _L), 1)
    row = lax.broadcasted_iota(jnp.int32, (_R, _L), 0)
    tri_l = (lax.broadcasted_iota(jnp.int32, (_L, _L), 0)
             < lax.broadcasted_iota(jnp.int32, (_L, _L), 1)).astype(jnp.float32)

    o = o_in_ref[...]
    os_, _ = _bitonic_desc(o, None, lane, row)
    tau = _extract(os_, (_K - 1) // _L, (_K - 1) % _L, lane, row)
    n2 = jnp.sum(jnp.where(o > tau, 1.0, 0.0)).astype(jnp.int32)
    need2 = _K - n2
    eq2 = o == tau
    eqr2 = _excl_cumsum(jnp.where(eq2, 1.0, 0.0), tri_l).astype(jnp.int32)
    sel = (o > tau) | (eq2 & (eqr2 < need2))
    sel_ref[...] = jnp.where(sel, 1, 0).astype(jnp.int32)


def _tc_stage(key, lp, g_arr):
    o, ksafe = pl.pallas_call(
        _k1a_body,
        out_shape=(
            jax.ShapeDtypeStruct((_R, _L), jnp.int32),
            jax.ShapeDtypeStruct((_R, _L), jnp.int32),
        ),
    )(key, lp, g_arr)
    sel = pl.pallas_call(
        _k1b_body,
        out_shape=jax.ShapeDtypeStruct((_R, _L), jnp.int32),
    )(o)
    return sel, ksafe


def _sc_gather(ksafe_flat, sel_flat):
    info = plsc.get_sparse_core_info()
    nw = info.num_cores * info.num_subcores
    chunk = _N // nw
    mesh = plsc.VectorSubcoreMesh(core_axis_name="c", subcore_axis_name="s")

    @functools.partial(
        pl.kernel,
        mesh=mesh,
        out_type=jax.ShapeDtypeStruct((_N,), jnp.int32),
        scratch_types=[
            pltpu.VMEM((chunk,), jnp.int32),
            pltpu.VMEM((chunk,), jnp.int32),
            pltpu.SemaphoreType.DMA,
        ],
    )
    def gath(idx_hbm, sel_hbm, out_hbm, idx_v, res_v, sem):
        wid = lax.axis_index("s") * info.num_cores + lax.axis_index("c")
        base = wid * chunk
        pltpu.sync_copy(idx_hbm.at[pl.ds(base, chunk)], idx_v)
        pltpu.async_copy(sel_hbm.at[idx_v], res_v, sem).wait()
        pltpu.sync_copy(res_v, out_hbm.at[pl.ds(base, chunk)])

    return gath(ksafe_flat, sel_flat)


def kernel(cam):
    c = cam + _EPS
    s = c.sum()
    p = (c / s).reshape(_N)
    lp = jnp.log(p)

    skey = jax.random.fold_in(jax.random.key(0), 1)
    g = jax.random.gumbel(skey, (_M,), dtype=jnp.float32)
    g_arr = jnp.concatenate([g, jnp.zeros((_N - _M,), jnp.float32)]).reshape(_R, _L)

    key = lax.bitcast_convert_type(p, jnp.int32).reshape(_R, _L)
    sel, ksafe = _tc_stage(key, lp.reshape(_R, _L), g_arr)

    out = _sc_gather(ksafe.reshape(_N), sel.reshape(_N))
    return out.reshape(_H, _W)
